# Initial kernel scaffold; baseline (speedup 1.0000x reference)
import jax
import jax.numpy as jnp
from jax.experimental import pallas as pl


def kernel(logits1, logits2, logits3, logits4, logits5, logits6, lbl, sample):
    raise NotImplementedError("write your pallas kernel here")


# speedup vs baseline: 1.0255x; 1.0255x over previous
import functools

import jax
import jax.numpy as jnp
from jax import lax
from jax.experimental import pallas as pl
from jax.experimental.pallas import tpu as pltpu
from jax.experimental.pallas import tpu_sc as plsc

A4 = 0.1
TEMP = 1.5
EPS = 1e-8

_NC = 2
_NS = 16
_NW = _NC * _NS


def _norm_body(l3_ref, l6_ref, out_ref):
    y = l3_ref[...] + A4 * l6_ref[...]
    ss = jnp.sum(y * y, axis=1, keepdims=True)
    n = jnp.maximum(jnp.sqrt(ss), EPS)
    out_ref[...] = y / n


def _normalize(l3p, l6p):
    npad, d = l3p.shape
    br = 256
    return pl.pallas_call(
        _norm_body,
        grid=(npad // br,),
        in_specs=[
            pl.BlockSpec((br, d), lambda i: (i, 0)),
            pl.BlockSpec((br, d), lambda i: (i, 0)),
        ],
        out_specs=pl.BlockSpec((br, d), lambda i: (i, 0)),
        out_shape=jax.ShapeDtypeStruct((npad, d), jnp.float32),
    )(l3p, l6p)


def _sc_gather(table, idx):
    b = idx.shape[0]
    d = table.shape[1]
    bpw = b // _NW
    k = 128
    nchunks = bpw // k
    mesh = plsc.VectorSubcoreMesh(core_axis_name="c", subcore_axis_name="s")

    @functools.partial(
        pl.kernel,
        mesh=mesh,
        out_type=jax.ShapeDtypeStruct((b, d), jnp.float32),
        scratch_types=[
            pltpu.VMEM((bpw,), jnp.int32),
            pltpu.VMEM((k, d), jnp.float32),
            pltpu.SemaphoreType.DMA,
        ],
    )
    def gather_kernel(table_hbm, idx_hbm, out_hbm, idx_v, rows_v, sem):
        wid = lax.axis_index("s") * _NC + lax.axis_index("c")
        base = wid * bpw
        pltpu.sync_copy(idx_hbm.at[pl.ds(base, bpw)], idx_v)
        for c in range(nchunks):
            pltpu.async_copy(
                table_hbm.at[idx_v.at[pl.ds(c * k, k)]], rows_v, sem
            ).wait()
            pltpu.sync_copy(rows_v, out_hbm.at[pl.ds(base + c * k, k)])

    return gather_kernel(table, idx)


def _loss_body(fhat_ref, gath_ref, l1_ref, l4_ref, l2_ref, l5_ref, lbl_ref,
               out_ref, acc_ref, *, br, s, n_real):
    i = pl.program_id(0)

    @pl.when(i == 0)
    def _():
        acc_ref[0] = 0.0

    a = fhat_ref[...]
    g = gath_ref[...]
    sims = jnp.sum(a[:, None, :] * g, axis=2)
    exp_s = jnp.exp(sims) / TEMP
    j = lax.broadcasted_iota(jnp.int32, (br, s), 1)
    num = jnp.sum(jnp.where(j == 0, exp_s, 0.0), axis=1, keepdims=True)
    den = jnp.sum(jnp.where(j > 0, exp_s, 0.0), axis=1, keepdims=True)
    res = -jnp.log(num / den)
    row = i * br + lax.broadcasted_iota(jnp.int32, (br, 1), 0)
    acc_ref[0] += jnp.sum(jnp.where(row < n_real, res, 0.0))

    @pl.when(i == pl.num_programs(0) - 1)
    def _():
        x1 = l1_ref[...] + A4 * l4_ref[...]
        x2 = l2_ref[...] + A4 * l5_ref[...]
        z = lbl_ref[...]
        b1 = jnp.mean(jnp.maximum(x1, 0.0) - x1 * z
                      + jnp.log1p(jnp.exp(-jnp.abs(x1))))
        b2 = jnp.mean(jnp.maximum(x2, 0.0) - x2 * z
                      + jnp.log1p(jnp.exp(-jnp.abs(x2))))
        total = b1 + b2 + acc_ref[0] / n_real
        out_ref[...] = jnp.broadcast_to(total, (1, 1))


def _loss(fhat, gath3, l1, l4, l2, l5, lbl, n_real):
    npad, s, d = gath3.shape
    br = 256
    k2 = l1.shape[1]
    small = pl.BlockSpec((1, k2), lambda i: (0, 0))
    return pl.pallas_call(
        functools.partial(_loss_body, br=br, s=s, n_real=n_real),
        grid=(npad // br,),
        in_specs=[
            pl.BlockSpec((br, d), lambda i: (i, 0)),
            pl.BlockSpec((br, s, d), lambda i: (i, 0, 0)),
            small, small, small, small, small,
        ],
        out_specs=pl.BlockSpec((1, 1), lambda i: (0, 0)),
        out_shape=jax.ShapeDtypeStruct((1, 1), jnp.float32),
        scratch_shapes=[pltpu.SMEM((1,), jnp.float32)],
    )(fhat, gath3, l1, l4, l2, l5, lbl)


def kernel(logits1, logits2, logits3, logits4, logits5, logits6, lbl, sample):
    n, d = logits3.shape
    s = sample.shape[1]
    npad = ((n + 319) // 320) * 320
    l3p = jnp.pad(logits3, ((0, npad - n), (0, 0)))
    l6p = jnp.pad(logits6, ((0, npad - n), (0, 0)))
    fhat = _normalize(l3p, l6p)
    samp = jnp.pad(sample, ((0, npad - n), (0, 0))).astype(jnp.int32)
    idx = samp.reshape(-1)
    gath = _sc_gather(fhat, idx)
    gath3 = gath.reshape(npad, s, d)
    out = _loss(fhat, gath3, logits1, logits4, logits2, logits5, lbl, n)
    return out[0, 0]
